# baseline (speedup 1.0000x reference)
import functools

import jax
import jax.numpy as jnp
from jax import lax
from jax.experimental import pallas as pl
from jax.experimental.pallas import tpu as pltpu
from jax.experimental.pallas import tpu_sc as plsc

B = 4
SEQ = 8192
D = 1024
NC = 2
NS = 16
NW = NC * NS
ROWS_W = SEQ // NW
R = 8
CHUNKS = ROWS_W // R
CELEMS = R * D
VECS = CELEMS // 16
NBUF = 3

_mesh = plsc.VectorSubcoreMesh(core_axis_name="c", subcore_axis_name="s")


@functools.partial(
    pl.kernel,
    out_type=jax.ShapeDtypeStruct((B * SEQ * D,), jnp.float32),
    mesh=_mesh,
    scratch_types=[
        [pltpu.VMEM((CELEMS,), jnp.float32) for _ in range(NBUF)],
        [pltpu.VMEM((B * CELEMS,), jnp.float32) for _ in range(NBUF)],
        [pltpu.SemaphoreType.DMA for _ in range(NBUF)],
        [pltpu.SemaphoreType.DMA for _ in range(NBUF)],
        [pltpu.SemaphoreType.DMA for _ in range(NBUF)],
    ],
)
def _sc_add(x_hbm, pos_hbm, out_hbm, pos_v, x_v, sem_p, sem_i, sem_o):
    wid = lax.axis_index("s") * NC + lax.axis_index("c")
    seq0 = wid * ROWS_W

    def start_pos(c):
        off = (seq0 + c * R) * D
        return pltpu.async_copy(pos_hbm.at[pl.ds(off, CELEMS)],
                                pos_v[c % NBUF], sem_p[c % NBUF])

    def start_in(c):
        off = (seq0 + c * R) * D
        return [pltpu.async_copy(
                    x_hbm.at[pl.ds(b * SEQ * D + off, CELEMS)],
                    x_v[c % NBUF].at[pl.ds(b * CELEMS, CELEMS)],
                    sem_i[c % NBUF])
                for b in range(B)]

    def start_out(c, b):
        off = b * SEQ * D + (seq0 + c * R) * D
        return pltpu.async_copy(x_v[c % NBUF].at[pl.ds(b * CELEMS, CELEMS)],
                                out_hbm.at[pl.ds(off, CELEMS)],
                                sem_o[c % NBUF])

    pos_h = {0: start_pos(0)}
    in_h = {0: start_in(0)}
    out_h = {}
    for c in range(CHUNKS):
        p = c % NBUF
        if c + 1 < CHUNKS:
            if c - 2 in out_h:
                for h in out_h.pop(c - 2):
                    h.wait()
            pos_h[c + 1] = start_pos(c + 1)
            in_h[c + 1] = start_in(c + 1)
        pos_h.pop(c).wait()
        for h in in_h.pop(c):
            h.wait()
        xb = x_v[p]
        for b in range(B):
            def add_b(i, _, xb=xb, pv=pos_v[p], base=b * CELEMS):
                xb[pl.ds(base + i * 16, 16)] = (
                    xb[pl.ds(base + i * 16, 16)] + pv[pl.ds(i * 16, 16)])
                return 0
            lax.fori_loop(0, VECS, add_b, 0, unroll=8)
            out_h.setdefault(c, []).append(start_out(c, b))
    for c in sorted(out_h):
        for h in out_h[c]:
            h.wait()


def kernel(x, pos_table):
    out = _sc_add(x.reshape(-1), pos_table.reshape(-1))
    return out.reshape(x.shape)


# speedup vs baseline: 1.2311x; 1.2311x over previous
import functools

import jax
import jax.numpy as jnp
from jax import lax
from jax.experimental import pallas as pl
from jax.experimental.pallas import tpu as pltpu
from jax.experimental.pallas import tpu_sc as plsc

B = 4
SEQ = 8192
D = 1024
NC = 2
NS = 16
NW = NC * NS
ROWS_W = SEQ // NW
R = 8
CHUNKS = ROWS_W // R
CELEMS = R * D
VECS = CELEMS // 16
NBUF = 3

_mesh = plsc.VectorSubcoreMesh(core_axis_name="c", subcore_axis_name="s")


@functools.partial(
    pl.kernel,
    out_type=jax.ShapeDtypeStruct((B * SEQ * D,), jnp.float32),
    mesh=_mesh,
    scratch_types=[
        [pltpu.VMEM((CELEMS,), jnp.float32) for _ in range(NBUF)],
        [pltpu.VMEM((B * CELEMS,), jnp.float32) for _ in range(NBUF)],
        [pltpu.SemaphoreType.DMA for _ in range(NBUF)],
        [pltpu.SemaphoreType.DMA for _ in range(NBUF)],
        [pltpu.SemaphoreType.DMA for _ in range(NBUF)],
    ],
)
def _sc_add(x_hbm, pos_hbm, out_hbm, pos_v, x_v, sem_p, sem_i, sem_o):
    wid = lax.axis_index("s") * NC + lax.axis_index("c")
    seq0 = wid * ROWS_W

    def start_pos(c):
        off = (seq0 + c * R) * D
        return pltpu.async_copy(pos_hbm.at[pl.ds(off, CELEMS)],
                                pos_v[c % NBUF], sem_p[c % NBUF])

    def start_in(c):
        off = (seq0 + c * R) * D
        return [pltpu.async_copy(
                    x_hbm.at[pl.ds(b * SEQ * D + off, CELEMS)],
                    x_v[c % NBUF].at[pl.ds(b * CELEMS, CELEMS)],
                    sem_i[c % NBUF])
                for b in range(B)]

    def start_out(c, b):
        off = b * SEQ * D + (seq0 + c * R) * D
        return pltpu.async_copy(x_v[c % NBUF].at[pl.ds(b * CELEMS, CELEMS)],
                                out_hbm.at[pl.ds(off, CELEMS)],
                                sem_o[c % NBUF])

    pos_h = {0: start_pos(0)}
    in_h = {0: start_in(0)}
    out_h = {}
    for c in range(CHUNKS):
        p = c % NBUF
        if c + 1 < CHUNKS:
            if c - 2 in out_h:
                for h in out_h.pop(c - 2):
                    h.wait()
            pos_h[c + 1] = start_pos(c + 1)
            in_h[c + 1] = start_in(c + 1)
        pos_h.pop(c).wait()
        for h in in_h.pop(c):
            h.wait()
        xb = x_v[p]
        for b in range(B):
            @plsc.parallel_loop(0, CELEMS, 16, unroll=8)
            def add_b(i, xb=xb, pv=pos_v[p], base=b * CELEMS):
                xb[pl.ds(base + i, 16)] = (
                    xb[pl.ds(base + i, 16)] + pv[pl.ds(i, 16)])
            out_h.setdefault(c, []).append(start_out(c, b))
    for c in sorted(out_h):
        for h in out_h[c]:
            h.wait()


def kernel(x, pos_table):
    out = _sc_add(x.reshape(-1), pos_table.reshape(-1))
    return out.reshape(x.shape)


# speedup vs baseline: 3.7578x; 3.0524x over previous
import functools

import jax
import jax.numpy as jnp
from jax import lax
from jax.experimental import pallas as pl
from jax.experimental.pallas import tpu as pltpu
from jax.experimental.pallas import tpu_sc as plsc

B = 4
SEQ = 8192
D = 1024
NC = 2
NS = 16
NW = NC * NS
ROWS_W = SEQ // NW
R = 16
CHUNKS = ROWS_W // R
CELEMS = R * D
STEPS = CHUNKS * B
NBX = 4
NBP = 2

_mesh = plsc.VectorSubcoreMesh(core_axis_name="c", subcore_axis_name="s")


@functools.partial(
    pl.kernel,
    out_type=jax.ShapeDtypeStruct((B * SEQ * D,), jnp.float32),
    mesh=_mesh,
    scratch_types=[
        [pltpu.VMEM((CELEMS,), jnp.float32) for _ in range(NBX)],
        [pltpu.VMEM((CELEMS,), jnp.float32) for _ in range(NBP)],
        [pltpu.SemaphoreType.DMA for _ in range(NBX)],
        [pltpu.SemaphoreType.DMA for _ in range(NBP)],
        [pltpu.SemaphoreType.DMA for _ in range(NBX)],
    ],
)
def _sc_add(x_hbm, pos_hbm, out_hbm, x_v, pos_v, sem_x, sem_p, sem_o):
    wid = lax.axis_index("s") * NC + lax.axis_index("c")
    seq0 = wid * ROWS_W

    def start_pos(c):
        off = (seq0 + c * R) * D
        return pltpu.async_copy(pos_hbm.at[pl.ds(off, CELEMS)],
                                pos_v[c % NBP], sem_p[c % NBP])

    def start_in(t):
        c, b = divmod(t, B)
        off = b * SEQ * D + (seq0 + c * R) * D
        return pltpu.async_copy(x_hbm.at[pl.ds(off, CELEMS)],
                                x_v[t % NBX], sem_x[t % NBX])

    def start_out(t):
        c, b = divmod(t, B)
        off = b * SEQ * D + (seq0 + c * R) * D
        return pltpu.async_copy(x_v[t % NBX],
                                out_hbm.at[pl.ds(off, CELEMS)],
                                sem_o[t % NBX])

    pos_h = {0: start_pos(0), 1: start_pos(1)}
    in_h = {0: start_in(0), 1: start_in(1)}
    out_h = {}
    for t in range(STEPS):
        c, b = divmod(t, B)
        if t + 2 < STEPS:
            if t - 2 in out_h:
                out_h.pop(t - 2).wait()
            in_h[t + 2] = start_in(t + 2)
        in_h.pop(t).wait()
        if b == 0:
            pos_h.pop(c).wait()

        @plsc.parallel_loop(0, CELEMS, 16, unroll=8)
        def add_b(i, xb=x_v[t % NBX], pv=pos_v[c % NBP]):
            plsc.addupdate(xb.at[pl.ds(i, 16)], pv[pl.ds(i, 16)])

        out_h[t] = start_out(t)
        if b == B - 1 and c + 2 < CHUNKS:
            pos_h[c + 2] = start_pos(c + 2)
    for t in sorted(out_h):
        out_h[t].wait()


def kernel(x, pos_table):
    out = _sc_add(x.reshape(-1), pos_table.reshape(-1))
    return out.reshape(x.shape)
